# baseline (speedup 1.0000x reference)
import functools

import jax
import jax.numpy as jnp
from jax import lax
from jax.experimental import pallas as pl
from jax.experimental.pallas import tpu as pltpu
from jax.experimental.pallas import tpu_sc as plsc

_VOCAB = 50257
_ADDED_OFF = 50257
_VQ_START = 50769
_EMBED = 1024
_VQ_DIM = 256
_VQ_VOCAB = 8192
_N_ADDED = 512
_EXT_ROWS = _N_ADDED + _VQ_VOCAB

_NC, _NS, _LANES = 2, 16, 16
_NW = _NC * _NS
_TOKENS = 4 * 8192
_PER_W = _TOKENS // _NW
_CHUNK = 64
_NCHUNK = _PER_W // _CHUNK
_DUMMY = _TOKENS
_OUT_ROWS = _TOKENS + 8


def _build_ext(W_add, W_cb, W_proj):
    def body(wadd_ref, wcb_ref, wproj_ref, out_ref):
        i = pl.program_id(0)

        @pl.when(i == 0)
        def _():
            out_ref[...] = wadd_ref[...]

        @pl.when(i > 0)
        def _():
            out_ref[...] = lax.dot_general(
                wcb_ref[...],
                wproj_ref[...],
                (((1,), (1,)), ((), ())),
                preferred_element_type=jnp.float32,
            )

    return pl.pallas_call(
        body,
        grid=(_EXT_ROWS // _N_ADDED,),
        in_specs=[
            pl.BlockSpec((_N_ADDED, _EMBED), lambda i: (0, 0)),
            pl.BlockSpec((_N_ADDED, _VQ_DIM), lambda i: (jnp.maximum(i - 1, 0), 0)),
            pl.BlockSpec((_EMBED, _VQ_DIM), lambda i: (0, 0)),
        ],
        out_specs=pl.BlockSpec((_N_ADDED, _EMBED), lambda i: (i, 0)),
        out_shape=jax.ShapeDtypeStruct((_EXT_ROWS, _EMBED), jnp.float32),
    )(W_add, W_cb, W_proj)


def _sc_lookup(x_flat, W_tok, W_ext):
    mesh = plsc.VectorSubcoreMesh(core_axis_name="c", subcore_axis_name="s")

    @functools.partial(
        pl.kernel,
        mesh=mesh,
        out_type=jax.ShapeDtypeStruct((_OUT_ROWS, _EMBED), jnp.float32),
        scratch_types=[
            pltpu.VMEM((_CHUNK,), jnp.int32),
            pltpu.VMEM((_CHUNK,), jnp.int32),
            pltpu.VMEM((_CHUNK,), jnp.int32),
            pltpu.VMEM((_CHUNK,), jnp.int32),
            pltpu.VMEM((_CHUNK, _EMBED), jnp.float32),
        ],
    )
    def k(x_hbm, tok_hbm, ext_hbm, out_hbm, xv, tokv, extv, dstv, rows):
        wid = lax.axis_index("s") * _NC + lax.axis_index("c")

        @pl.loop(0, _NCHUNK)
        def _(c):
            base = pl.multiple_of(wid * _PER_W + c * _CHUNK, _CHUNK)
            pltpu.sync_copy(x_hbm.at[pl.ds(base, _CHUNK)], xv)
            for j in range(_CHUNK // _LANES):
                v = xv[pl.ds(_LANES * j, _LANES)]
                is_text = v < _ADDED_OFF
                tokv[pl.ds(_LANES * j, _LANES)] = jnp.where(is_text, v, 0)
                extv[pl.ds(_LANES * j, _LANES)] = jnp.where(
                    is_text, 0, v - _ADDED_OFF
                )
                pos = base + _LANES * j + lax.iota(jnp.int32, _LANES)
                dstv[pl.ds(_LANES * j, _LANES)] = jnp.where(is_text, _DUMMY, pos)
            pltpu.sync_copy(tok_hbm.at[tokv], rows)
            pltpu.sync_copy(rows, out_hbm.at[pl.ds(base, _CHUNK)])
            pltpu.sync_copy(ext_hbm.at[extv], rows)
            pltpu.sync_copy(rows, out_hbm.at[dstv])

    return k(x_flat, W_tok, W_ext)


def kernel(x, W_tok, W_add, W_cb, W_proj):
    W_ext = _build_ext(W_add, W_cb, W_proj)
    out = _sc_lookup(x.reshape(-1), W_tok, W_ext)
    return out[:_TOKENS].reshape(x.shape + (_EMBED,))
